# baseline (speedup 1.0000x reference)
import functools

import jax
import jax.numpy as jnp
from jax import lax
from jax.experimental import pallas as pl
from jax.experimental.pallas import tpu as pltpu
from jax.experimental.pallas import tpu_sc as plsc

NC = 2
NS = 16
NW = NC * NS
LANES = 16

CH = 256
GATHER = 128


def _emb_body(x_hbm, tok_hbm, pos_hbm, out_hbm, idx_v, rows_v, pos_v, sem,
              *, T, E, rows_per_w):
    wid = lax.axis_index("s") * NC + lax.axis_index("c")
    pltpu.sync_copy(x_hbm.at[wid], idx_v)

    n_chunks = T // CH
    n_gather = CH // GATHER
    b_per_w = rows_per_w // T

    def chunk_body(c, _):
        pltpu.sync_copy(pos_hbm.at[pl.ds(c * CH, CH)], pos_v)
        for bl in range(b_per_w):
            cps = [
                pltpu.async_copy(
                    tok_hbm.at[idx_v.at[bl * (T // GATHER) + c * n_gather + j]],
                    rows_v.at[pl.ds(j * GATHER, GATHER)],
                    sem,
                )
                for j in range(n_gather)
            ]
            for cp in cps:
                cp.wait()

            def add_body(r, _):
                for k in range(E // LANES):
                    pv = pos_v[r, pl.ds(k * LANES, LANES)]
                    plsc.addupdate(rows_v.at[r, pl.ds(k * LANES, LANES)], pv)
                return 0

            lax.fori_loop(0, CH, add_body, 0, unroll=4)

            row0 = (wid * b_per_w + bl) * T + c * CH
            pltpu.sync_copy(rows_v, out_hbm.at[pl.ds(row0, CH)])
        return 0

    lax.fori_loop(0, n_chunks, chunk_body, 0)


def kernel(x, token_embedding_table, position_embedding_table):
    B, T = x.shape
    V, E = token_embedding_table.shape
    rows_per_w = (B * T) // NW
    x32 = x.astype(jnp.int32).reshape(NW, rows_per_w // GATHER, GATHER)

    mesh = plsc.VectorSubcoreMesh(core_axis_name="c", subcore_axis_name="s")
    body = functools.partial(_emb_body, T=T, E=E, rows_per_w=rows_per_w)
    run = pl.kernel(
        body,
        out_type=jax.ShapeDtypeStruct((B * T, E), jnp.float32),
        mesh=mesh,
        scratch_types=[
            pltpu.VMEM((rows_per_w // GATHER, GATHER), jnp.int32),
            pltpu.VMEM((CH, E), jnp.float32),
            pltpu.VMEM((CH, E), jnp.float32),
            pltpu.SemaphoreType.DMA,
        ],
        compiler_params=pltpu.CompilerParams(use_tc_tiling_on_sc=False),
    )
    out = run(x32, token_embedding_table, position_embedding_table)
    return out.reshape(B, T, E)
